# baseline (speedup 1.0000x reference)
import jax
import jax.numpy as jnp
from jax.experimental import pallas as pl
from jax.experimental.pallas import tpu as pltpu

DIM = 768
SEQ = 4096
NE = 1024
NC = 256
WIN = 4
NEG = -1e30
S_DELTA = 0.25
B_DELTA = 0.2
CROSS_DELTA = 0.15
HI = jax.lax.Precision.HIGHEST

WTILE = 512
ETILE = 128
STILE = 1024


def _bf(x):
    return x.astype(jnp.bfloat16).astype(jnp.float32)


def _ln(x, w, b):
    mu = jnp.mean(x, axis=-1, keepdims=True)
    var = jnp.mean((x - mu) ** 2, axis=-1, keepdims=True)
    return (x - mu) * jax.lax.rsqrt(var + 1e-5) * w + b


def _dot3(a, b):
    a_hi = a.astype(jnp.bfloat16)
    a_lo = (a - a_hi.astype(jnp.float32)).astype(jnp.bfloat16)
    b_hi = b.astype(jnp.bfloat16)
    b_lo = (b - b_hi.astype(jnp.float32)).astype(jnp.bfloat16)
    dims = (((1,), (0,)), ((), ()))
    out = jax.lax.dot_general(a_hi, b_hi, dims, preferred_element_type=jnp.float32)
    out = out + jax.lax.dot_general(a_hi, b_lo, dims, preferred_element_type=jnp.float32)
    out = out + jax.lax.dot_general(a_lo, b_hi, dims, preferred_element_type=jnp.float32)
    return out


def _topk_dsdv(q, src_s, src_v, state_hi=True, state_3x=False):
    scores = jax.lax.dot_general(
        q.astype(jnp.bfloat16), src_s.astype(jnp.bfloat16),
        (((1,), (1,)), ((), ())), preferred_element_type=jnp.float32)
    p, inv = _top4_p(scores)
    if state_3x:
        d_s = _dot3(p, src_s) * inv
    elif state_hi:
        d_s = jax.lax.dot_general(p, src_s, (((1,), (0,)), ((), ())),
                                  precision=HI) * inv
    else:
        d_s = jax.lax.dot_general(
            p.astype(jnp.bfloat16), src_s.astype(jnp.bfloat16),
            (((1,), (0,)), ((), ())), preferred_element_type=jnp.float32) * inv
    d_v = jax.lax.dot_general(
        p.astype(jnp.bfloat16), src_v.astype(jnp.bfloat16),
        (((1,), (0,)), ((), ())), preferred_element_type=jnp.float32) * inv
    return d_s, d_v


def _window_body(ps_ref, cs_ref, ns_ref, pv_ref, cv_ref, nv_ref, w_ref,
                 os_ref, ov_ref, osb_ref, ovb_ref):
    i = pl.program_id(0)
    base = i * WTILE
    s_c = cs_ref[...]
    v_c = cv_ref[...]
    a_s = jnp.concatenate(
        [ps_ref[WTILE - 8:, :], s_c, ns_ref[:8, :]], axis=0)
    a_v = jnp.concatenate(
        [pv_ref[WTILE - 8:, :], v_c, nv_ref[:8, :]], axis=0)
    qb = (s_c * w_ref[...]).astype(jnp.bfloat16)
    asb = a_s.astype(jnp.bfloat16)
    avb = a_v.astype(jnp.bfloat16)
    scores = jax.lax.dot_general(qb, asb, (((1,), (1,)), ((), ())),
                                 preferred_element_type=jnp.float32)
    li = jax.lax.broadcasted_iota(jnp.int32, (WTILE, WTILE + 16), 0)
    lj = jax.lax.broadcasted_iota(jnp.int32, (WTILE, WTILE + 16), 1)
    src = base + lj - 8
    valid = (jnp.abs(li - (lj - 8)) <= WIN) & (src >= 0) & (src < SEQ)
    scores = jnp.where(valid, scores, NEG)
    m = jnp.max(scores, axis=1, keepdims=True)
    e = jnp.exp(scores - m)
    attn = (e / jnp.sum(e, axis=1, keepdims=True)).astype(jnp.bfloat16)
    dims = (((1,), (0,)), ((), ()))
    acc_s = jax.lax.dot_general(attn, asb, dims,
                                preferred_element_type=jnp.float32)
    acc_v = jax.lax.dot_general(attn, avb, dims,
                                preferred_element_type=jnp.float32)
    new_s = s_c + S_DELTA * acc_s
    new_v = v_c + S_DELTA * acc_v
    os_ref[...] = new_s
    ov_ref[...] = new_v
    osb_ref[...] = new_s.astype(jnp.bfloat16)
    ovb_ref[...] = new_v.astype(jnp.bfloat16)


def _top4_p(scores):
    v1 = jnp.max(scores, axis=1, keepdims=True)
    s = jnp.where(scores == v1, NEG, scores)
    v2 = jnp.max(s, axis=1, keepdims=True)
    s = jnp.where(s == v2, NEG, s)
    v3 = jnp.max(s, axis=1, keepdims=True)
    s = jnp.where(s == v3, NEG, s)
    v4 = jnp.max(s, axis=1, keepdims=True)
    p = jnp.where(scores >= v4, jnp.exp(scores - v1), 0.0)
    inv = 1.0 / jnp.sum(p, axis=1, keepdims=True)
    return p, inv


EHALF = NE // 2


def _mid_body(pe_ref, ss_ref, ssb_ref, svb_ref, pc_ref, ws2b_ref, wpe_ref,
              we2c_ref, wpc_ref, lnew_ref, lneb_ref, lncw_ref, lncb_ref,
              cs_ref, cv_ref, es_scr, ev_scr, es2_scr, ev2_scr):
    pid = pl.program_id(0)
    n_e = NE // ETILE

    @pl.when(pid < n_e)
    def _():
        r0 = pid * ETILE
        e0 = pe_ref[pl.ds(r0, ETILE), :]
        q = (e0 * ws2b_ref[...]).astype(jnp.bfloat16)
        scores = jax.lax.dot_general(q, ssb_ref[...],
                                     (((1,), (1,)), ((), ())),
                                     preferred_element_type=jnp.float32)
        p, inv = _top4_p(scores)
        d_s = jax.lax.dot_general(p, ss_ref[...], (((1,), (0,)), ((), ())),
                                  precision=HI) * inv
        d_v = jax.lax.dot_general(p.astype(jnp.bfloat16), svb_ref[...],
                                  (((1,), (0,)), ((), ())),
                                  preferred_element_type=jnp.float32) * inv
        es_scr[pl.ds(r0, ETILE), :] = e0 + CROSS_DELTA * d_s
        ev_scr[pl.ds(r0, ETILE), :] = e0 + CROSS_DELTA * d_v

    for half in range(2):
        @pl.when(pid == n_e + half)
        def _(half=half):
            r0 = half * EHALF
            es_t = es_scr[pl.ds(r0, EHALF), :]
            ev_t = ev_scr[pl.ds(r0, EHALF), :]
            d_s, d_v = _topk_dsdv(es_t * wpe_ref[...], es_scr[...],
                                  ev_scr[...])
            es2_scr[pl.ds(r0, EHALF), :] = es_t + B_DELTA * d_s
            ev2_scr[pl.ds(r0, EHALF), :] = _ln(
                ev_t + B_DELTA * d_v, lnew_ref[...], lneb_ref[...])

    @pl.when(pid == n_e + 2)
    def _():
        c0 = pc_ref[...]
        d_s, d_v = _topk_dsdv(c0 * we2c_ref[...], es2_scr[...],
                              ev2_scr[...])
        cs = c0 + CROSS_DELTA * d_s
        cv = c0 + CROSS_DELTA * d_v
        d_s, d_v = _topk_dsdv(cs * wpc_ref[...], cs, cv)
        cs_ref[...] = cs + B_DELTA * d_s
        cv_ref[...] = _ln(cv + B_DELTA * d_v, lncw_ref[...], lncb_ref[...])


def _b2s_body(ss_ref, sv_ref, cs_ref, cv_ref, w_ref, lnw_ref, lnb_ref,
              o_ref):
    s0 = ss_ref[...]
    v0 = sv_ref[...]
    d_s, d_v = _topk_dsdv(s0 * w_ref[...], cs_ref[...], cv_ref[...],
                          state_hi=False)
    o_ref[0] = jnp.tanh(s0 + CROSS_DELTA * d_s)
    o_ref[1] = _ln(v0 + CROSS_DELTA * d_v, lnw_ref[...], lnb_ref[...])


def _full(shape):
    return pl.BlockSpec(shape, lambda i: (0, 0))


def _f32(shape):
    return jax.ShapeDtypeStruct(shape, jnp.float32)


def kernel(s_state, s_val, w_pair_s, w_pair_e, w_pair_c, w_route_s2b,
           w_route_e2c, w_route_b2s, pos_e, pos_c, ln_s_w, ln_s_b,
           ln_e_w, ln_e_b, ln_c_w, ln_c_b):
    s2 = s_state.reshape(SEQ, DIM)
    v2 = s_val.reshape(SEQ, DIM)
    wps = w_pair_s.reshape(1, DIM)
    wpe = w_pair_e.reshape(1, DIM)
    wpc = w_pair_c.reshape(1, DIM)
    ws2b = w_route_s2b.reshape(1, DIM)
    we2c = w_route_e2c.reshape(1, DIM)
    wb2s = w_route_b2s.reshape(1, DIM)
    lnsw = ln_s_w.reshape(1, DIM)
    lnsb = ln_s_b.reshape(1, DIM)
    lnew = ln_e_w.reshape(1, DIM)
    lneb = ln_e_b.reshape(1, DIM)
    lncw = ln_c_w.reshape(1, DIM)
    lncb = ln_c_b.reshape(1, DIM)

    nb = SEQ // WTILE
    tile = lambda m: pl.BlockSpec((WTILE, DIM), m)
    prev_m = lambda i: (jnp.maximum(i - 1, 0), 0)
    next_m = lambda i: (jnp.minimum(i + 1, nb - 1), 0)
    cur_m = lambda i: (i, 0)
    s1, v1, s1b, v1b = pl.pallas_call(
        _window_body,
        grid=(nb,),
        in_specs=[tile(prev_m), tile(cur_m), tile(next_m),
                  tile(prev_m), tile(cur_m), tile(next_m),
                  _full((1, DIM))],
        out_specs=[pl.BlockSpec((WTILE, DIM), lambda i: (i, 0))] * 4,
        out_shape=[_f32((SEQ, DIM))] * 2
        + [jax.ShapeDtypeStruct((SEQ, DIM), jnp.bfloat16)] * 2,
    )(s2, s2, s2, v2, v2, v2, wps)

    c_s, c_v = pl.pallas_call(
        _mid_body,
        grid=(NE // ETILE + 3,),
        in_specs=[_full((NE, DIM)), _full((SEQ, DIM)), _full((SEQ, DIM)),
                  _full((SEQ, DIM)), _full((NC, DIM))]
        + [_full((1, DIM))] * 8,
        out_specs=[_full((NC, DIM))] * 2,
        out_shape=[_f32((NC, DIM))] * 2,
        scratch_shapes=[pltpu.VMEM((NE, DIM), jnp.float32)] * 4,
    )(pos_e, s1, s1b, v1b, pos_c, ws2b, wpe, we2c, wpc,
      lnew, lneb, lncw, lncb)

    out = pl.pallas_call(
        _b2s_body,
        grid=(SEQ // STILE,),
        in_specs=[pl.BlockSpec((STILE, DIM), lambda i: (i, 0)),
                  pl.BlockSpec((STILE, DIM), lambda i: (i, 0)),
                  _full((NC, DIM)), _full((NC, DIM)),
                  pl.BlockSpec((1, DIM), lambda i: (0, 0)),
                  pl.BlockSpec((1, DIM), lambda i: (0, 0)),
                  pl.BlockSpec((1, DIM), lambda i: (0, 0))],
        out_specs=pl.BlockSpec((2, STILE, DIM), lambda i: (0, i, 0)),
        out_shape=jax.ShapeDtypeStruct((2, SEQ, DIM), jnp.float32),
    )(s1, v1, c_s, c_v, wb2s, lnsw, lnsb)

    return out.reshape(2, 1, SEQ, DIM)


# speedup vs baseline: 1.0554x; 1.0554x over previous
import jax
import jax.numpy as jnp
from jax.experimental import pallas as pl
from jax.experimental.pallas import tpu as pltpu

DIM = 768
SEQ = 4096
NE = 1024
NC = 256
WIN = 4
NEG = -1e30
S_DELTA = 0.25
B_DELTA = 0.2
CROSS_DELTA = 0.15
HI = jax.lax.Precision.HIGHEST

WTILE = 512
ETILE = 256
STILE = 1024


def _bf(x):
    return x.astype(jnp.bfloat16).astype(jnp.float32)


def _ln(x, w, b):
    mu = jnp.mean(x, axis=-1, keepdims=True)
    var = jnp.mean((x - mu) ** 2, axis=-1, keepdims=True)
    return (x - mu) * jax.lax.rsqrt(var + 1e-5) * w + b


def _dot3(a, b):
    a_hi = a.astype(jnp.bfloat16)
    a_lo = (a - a_hi.astype(jnp.float32)).astype(jnp.bfloat16)
    b_hi = b.astype(jnp.bfloat16)
    b_lo = (b - b_hi.astype(jnp.float32)).astype(jnp.bfloat16)
    dims = (((1,), (0,)), ((), ()))
    out = jax.lax.dot_general(a_hi, b_hi, dims, preferred_element_type=jnp.float32)
    out = out + jax.lax.dot_general(a_hi, b_lo, dims, preferred_element_type=jnp.float32)
    out = out + jax.lax.dot_general(a_lo, b_hi, dims, preferred_element_type=jnp.float32)
    return out


def _topk_dsdv(q, src_s, src_v, state_hi=True, state_3x=False):
    scores = jax.lax.dot_general(
        q.astype(jnp.bfloat16), src_s.astype(jnp.bfloat16),
        (((1,), (1,)), ((), ())), preferred_element_type=jnp.float32)
    p, inv = _top4_p(scores)
    if state_3x:
        d_s = _dot3(p, src_s) * inv
    elif state_hi:
        d_s = jax.lax.dot_general(p, src_s, (((1,), (0,)), ((), ())),
                                  precision=HI) * inv
    else:
        d_s = jax.lax.dot_general(
            p.astype(jnp.bfloat16), src_s.astype(jnp.bfloat16),
            (((1,), (0,)), ((), ())), preferred_element_type=jnp.float32) * inv
    d_v = jax.lax.dot_general(
        p.astype(jnp.bfloat16), src_v.astype(jnp.bfloat16),
        (((1,), (0,)), ((), ())), preferred_element_type=jnp.float32) * inv
    return d_s, d_v


def _window_body(ps_ref, cs_ref, ns_ref, pv_ref, cv_ref, nv_ref, w_ref,
                 os_ref, ov_ref, osb_ref, ovb_ref):
    i = pl.program_id(0)
    base = i * WTILE
    s_c = cs_ref[...]
    v_c = cv_ref[...]
    a_s = jnp.concatenate(
        [ps_ref[WTILE - 8:, :], s_c, ns_ref[:8, :]], axis=0)
    a_v = jnp.concatenate(
        [pv_ref[WTILE - 8:, :], v_c, nv_ref[:8, :]], axis=0)
    qb = (s_c * w_ref[...]).astype(jnp.bfloat16)
    asb = a_s.astype(jnp.bfloat16)
    avb = a_v.astype(jnp.bfloat16)
    scores = jax.lax.dot_general(qb, asb, (((1,), (1,)), ((), ())),
                                 preferred_element_type=jnp.float32)
    li = jax.lax.broadcasted_iota(jnp.int32, (WTILE, WTILE + 16), 0)
    lj = jax.lax.broadcasted_iota(jnp.int32, (WTILE, WTILE + 16), 1)
    src = base + lj - 8
    valid = (jnp.abs(li - (lj - 8)) <= WIN) & (src >= 0) & (src < SEQ)
    scores = jnp.where(valid, scores, NEG)
    m = jnp.max(scores, axis=1, keepdims=True)
    e = jnp.exp(scores - m)
    attn = (e / jnp.sum(e, axis=1, keepdims=True)).astype(jnp.bfloat16)
    dims = (((1,), (0,)), ((), ()))
    acc_s = jax.lax.dot_general(attn, asb, dims,
                                preferred_element_type=jnp.float32)
    acc_v = jax.lax.dot_general(attn, avb, dims,
                                preferred_element_type=jnp.float32)
    new_s = s_c + S_DELTA * acc_s
    new_v = v_c + S_DELTA * acc_v
    os_ref[...] = new_s
    ov_ref[...] = new_v
    osb_ref[...] = new_s.astype(jnp.bfloat16)
    ovb_ref[...] = new_v.astype(jnp.bfloat16)


def _top4_p(scores, split=False):
    if split:
        h = scores.shape[1] // 2
        cols = []
        for sl in (scores[:, :h], scores[:, h:]):
            s = sl
            for _ in range(3):
                v = jnp.max(s, axis=1, keepdims=True)
                cols.append(v)
                s = jnp.where(s == v, NEG, s)
            cols.append(jnp.max(s, axis=1, keepdims=True))
        cand = jnp.concatenate(cols, axis=1)
        v1 = jnp.max(cand, axis=1, keepdims=True)
        s = cand
        for _ in range(3):
            v = jnp.max(s, axis=1, keepdims=True)
            s = jnp.where(s == v, NEG, s)
        v4 = jnp.max(s, axis=1, keepdims=True)
    else:
        v1 = jnp.max(scores, axis=1, keepdims=True)
        s = jnp.where(scores == v1, NEG, scores)
        v2 = jnp.max(s, axis=1, keepdims=True)
        s = jnp.where(s == v2, NEG, s)
        v3 = jnp.max(s, axis=1, keepdims=True)
        s = jnp.where(s == v3, NEG, s)
        v4 = jnp.max(s, axis=1, keepdims=True)
    p = jnp.where(scores >= v4, jnp.exp(scores - v1), 0.0)
    inv = 1.0 / jnp.sum(p, axis=1, keepdims=True)
    return p, inv


def _s2b_body(pe_ref, ss_ref, ssb_ref, svb_ref, w_ref, os_ref, ov_ref):
    e0 = pe_ref[...]
    q = (e0 * w_ref[...]).astype(jnp.bfloat16)
    scores = jax.lax.dot_general(q, ssb_ref[...], (((1,), (1,)), ((), ())),
                                 preferred_element_type=jnp.float32)
    p, inv = _top4_p(scores)
    d_s = jax.lax.dot_general(p, ss_ref[...], (((1,), (0,)), ((), ())),
                              precision=HI) * inv
    d_v = jax.lax.dot_general(p.astype(jnp.bfloat16), svb_ref[...],
                              (((1,), (0,)), ((), ())),
                              preferred_element_type=jnp.float32) * inv
    os_ref[...] = e0 + CROSS_DELTA * d_s
    ov_ref[...] = e0 + CROSS_DELTA * d_v


def _e_body(es_ref, ev_ref, w_ref, lnw_ref, lnb_ref, os_ref, ov_ref):
    es = es_ref[...]
    ev = ev_ref[...]
    d_s, d_v = _topk_dsdv(es * w_ref[...], es, ev)
    os_ref[...] = es + B_DELTA * d_s
    ov_ref[...] = _ln(ev + B_DELTA * d_v, lnw_ref[...], lnb_ref[...])


def _c_body(pc_ref, es_ref, ev_ref, we2c_ref, wpc_ref, lnw_ref, lnb_ref,
            os_ref, ov_ref):
    c0 = pc_ref[...]
    d_s, d_v = _topk_dsdv(c0 * we2c_ref[...], es_ref[...], ev_ref[...])
    cs = c0 + CROSS_DELTA * d_s
    cv = c0 + CROSS_DELTA * d_v
    d_s, d_v = _topk_dsdv(cs * wpc_ref[...], cs, cv)
    os_ref[...] = cs + B_DELTA * d_s
    ov_ref[...] = _ln(cv + B_DELTA * d_v, lnw_ref[...], lnb_ref[...])


def _b2s_body(ss_ref, sv_ref, cs_ref, cv_ref, w_ref, lnw_ref, lnb_ref,
              o_ref):
    s0 = ss_ref[...]
    v0 = sv_ref[...]
    d_s, d_v = _topk_dsdv(s0 * w_ref[...], cs_ref[...], cv_ref[...],
                          state_hi=False)
    o_ref[0] = jnp.tanh(s0 + CROSS_DELTA * d_s)
    o_ref[1] = _ln(v0 + CROSS_DELTA * d_v, lnw_ref[...], lnb_ref[...])


def _full(shape):
    return pl.BlockSpec(shape, lambda i: (0, 0))


def _f32(shape):
    return jax.ShapeDtypeStruct(shape, jnp.float32)


def kernel(s_state, s_val, w_pair_s, w_pair_e, w_pair_c, w_route_s2b,
           w_route_e2c, w_route_b2s, pos_e, pos_c, ln_s_w, ln_s_b,
           ln_e_w, ln_e_b, ln_c_w, ln_c_b):
    s2 = s_state.reshape(SEQ, DIM)
    v2 = s_val.reshape(SEQ, DIM)
    wps = w_pair_s.reshape(1, DIM)
    wpe = w_pair_e.reshape(1, DIM)
    wpc = w_pair_c.reshape(1, DIM)
    ws2b = w_route_s2b.reshape(1, DIM)
    we2c = w_route_e2c.reshape(1, DIM)
    wb2s = w_route_b2s.reshape(1, DIM)
    lnsw = ln_s_w.reshape(1, DIM)
    lnsb = ln_s_b.reshape(1, DIM)
    lnew = ln_e_w.reshape(1, DIM)
    lneb = ln_e_b.reshape(1, DIM)
    lncw = ln_c_w.reshape(1, DIM)
    lncb = ln_c_b.reshape(1, DIM)

    nb = SEQ // WTILE
    tile = lambda m: pl.BlockSpec((WTILE, DIM), m)
    prev_m = lambda i: (jnp.maximum(i - 1, 0), 0)
    next_m = lambda i: (jnp.minimum(i + 1, nb - 1), 0)
    cur_m = lambda i: (i, 0)
    s1, v1, s1b, v1b = pl.pallas_call(
        _window_body,
        grid=(nb,),
        in_specs=[tile(prev_m), tile(cur_m), tile(next_m),
                  tile(prev_m), tile(cur_m), tile(next_m),
                  _full((1, DIM))],
        out_specs=[pl.BlockSpec((WTILE, DIM), lambda i: (i, 0))] * 4,
        out_shape=[_f32((SEQ, DIM))] * 2
        + [jax.ShapeDtypeStruct((SEQ, DIM), jnp.bfloat16)] * 2,
    )(s2, s2, s2, v2, v2, v2, wps)

    e_s, e_v = pl.pallas_call(
        _s2b_body,
        grid=(NE // ETILE,),
        in_specs=[pl.BlockSpec((ETILE, DIM), lambda i: (i, 0)),
                  _full((SEQ, DIM)), _full((SEQ, DIM)), _full((SEQ, DIM)),
                  _full((1, DIM))],
        out_specs=[pl.BlockSpec((ETILE, DIM), lambda i: (i, 0))] * 2,
        out_shape=[_f32((NE, DIM))] * 2,
    )(pos_e, s1, s1b, v1b, ws2b)

    e_s, e_v = pl.pallas_call(
        _e_body,
        out_shape=[_f32((NE, DIM))] * 2,
    )(e_s, e_v, wpe, lnew, lneb)

    c_s, c_v = pl.pallas_call(
        _c_body,
        out_shape=[_f32((NC, DIM))] * 2,
    )(pos_c, e_s, e_v, we2c, wpc, lncw, lncb)

    out = pl.pallas_call(
        _b2s_body,
        grid=(SEQ // STILE,),
        in_specs=[pl.BlockSpec((STILE, DIM), lambda i: (i, 0)),
                  pl.BlockSpec((STILE, DIM), lambda i: (i, 0)),
                  _full((NC, DIM)), _full((NC, DIM)),
                  pl.BlockSpec((1, DIM), lambda i: (0, 0)),
                  pl.BlockSpec((1, DIM), lambda i: (0, 0)),
                  pl.BlockSpec((1, DIM), lambda i: (0, 0))],
        out_specs=pl.BlockSpec((2, STILE, DIM), lambda i: (0, i, 0)),
        out_shape=jax.ShapeDtypeStruct((2, SEQ, DIM), jnp.float32),
    )(s1, v1, c_s, c_v, wb2s, lnsw, lnsb)

    return out.reshape(2, 1, SEQ, DIM)
